# baseline (speedup 1.0000x reference)
import functools

import jax
import jax.numpy as jnp
from jax import lax
from jax.experimental import pallas as pl
from jax.experimental.pallas import tpu as pltpu
from jax.experimental.pallas import tpu_sc as plsc

N = 10000
NP = 10240
E = 320000
D = 128
NC = 2
NS = 16
NW = NC * NS
EPW = 10240
EPAD = NW * EPW
EC = 64
PH = 4
CPP = EPW // PH // EC
P = 20000
PP = 20480
PPW = PP // NW
PCH = PPW // 128
RPS = NP // NS

_MESH = dict(core_axis_name="c", subcore_axis_name="s", num_cores=NC,
             num_subcores=NS)


def _deg_body(dst_hbm, out_hbm, idx_v, acc_v, part_v, res_v, stage_sh):
    cid = lax.axis_index("c")
    sid = lax.axis_index("s")
    wid = cid * NS + sid

    def zero(i, c):
        acc_v[pl.ds(i * 16, 16)] = jnp.zeros((16,), jnp.float32)
        return c
    lax.fori_loop(0, NP // 16, zero, 0)

    pltpu.sync_copy(dst_hbm.at[pl.ds(wid * EPW, EPW)], idx_v)

    ones = jnp.ones((16,), jnp.float32)

    def step(i, c):
        iv = idx_v[pl.ds(i * 16, 16)]
        plsc.addupdate_scatter(acc_v, [iv], ones)
        return c
    lax.fori_loop(0, EPW // 16, step, 0)

    pltpu.sync_copy(acc_v, stage_sh.at[sid])
    plsc.subcore_barrier()

    r0 = sid * RPS
    for t in range(NS):
        pltpu.sync_copy(stage_sh.at[t, pl.ds(r0, RPS)], part_v.at[t])

    def red(j, c):
        sl = pl.ds(j * 16, 16)
        s = part_v[0, sl]
        for t in range(1, NS):
            s = s + part_v[t, sl]
        res_v[sl] = s
        return c
    lax.fori_loop(0, RPS // 16, red, 0)

    pltpu.sync_copy(res_v, out_hbm.at[cid, pl.ds(r0, RPS)])


def _deg_call(dst_flat):
    k = functools.partial(
        pl.kernel,
        out_type=jax.ShapeDtypeStruct((NC, NP), jnp.float32),
        mesh=plsc.VectorSubcoreMesh(**_MESH),
        compiler_params=pltpu.CompilerParams(needs_layout_passes=False),
        scratch_types=[
            pltpu.VMEM((EPW,), jnp.int32),
            pltpu.VMEM((NP,), jnp.float32),
            pltpu.VMEM((NS, RPS), jnp.float32),
            pltpu.VMEM((RPS,), jnp.float32),
            pltpu.VMEM_SHARED((NS, NP), jnp.float32),
        ],
    )(_deg_body)
    return k(dst_flat)


NBUF = 4
GDEPTH = 3
SLAG = 1


def _spmm_body(hs_hbm, si_hbm, di_hbm, z_hbm, out_hbm,
               sidx_v, didx_v, rows_v, acc_sh, gsem, ssem):
    cid = lax.axis_index("c")
    sid = lax.axis_index("s")
    wid = cid * NS + sid
    r0 = sid * RPS

    pltpu.sync_copy(z_hbm.at[pl.ds(r0, RPS)], acc_sh.at[pl.ds(r0, RPS)])
    plsc.subcore_barrier()

    def phase(p, carry0):
        pltpu.sync_copy(si_hbm.at[wid, p], sidx_v)
        pltpu.sync_copy(di_hbm.at[wid, p], didx_v)

        for b in range(GDEPTH):
            pltpu.async_copy(hs_hbm.at[sidx_v.at[b]], rows_v.at[b], gsem)

        def step(i, carry):
            b = lax.rem(i, NBUF)
            pltpu.make_async_copy(hs_hbm.at[pl.ds(0, EC)], rows_v.at[b],
                                  gsem).wait()
            pltpu.async_copy(rows_v.at[b], acc_sh.at[didx_v.at[i]], ssem,
                             add=True)

            @pl.when(i >= SLAG)
            def _():
                pltpu.make_async_copy(rows_v.at[0], acc_sh.at[pl.ds(0, EC)],
                                      ssem).wait()

            @pl.when(i + GDEPTH < CPP)
            def _():
                b2 = lax.rem(i + GDEPTH, NBUF)
                pltpu.async_copy(hs_hbm.at[sidx_v.at[i + GDEPTH]],
                                 rows_v.at[b2], gsem)
            return carry
        lax.fori_loop(0, CPP, step, 0)

        for _ in range(SLAG):
            pltpu.make_async_copy(rows_v.at[0], acc_sh.at[pl.ds(0, EC)],
                                  ssem).wait()
        return carry0
    lax.fori_loop(0, PH, phase, 0)

    plsc.subcore_barrier()
    pltpu.sync_copy(acc_sh.at[pl.ds(r0, RPS)], out_hbm.at[cid, pl.ds(r0, RPS)])


def _spmm_call(hs, src4, dst4, zeros2d):
    k = functools.partial(
        pl.kernel,
        out_type=jax.ShapeDtypeStruct((NC, NP, D), jnp.float32),
        mesh=plsc.VectorSubcoreMesh(**_MESH),
        compiler_params=pltpu.CompilerParams(needs_layout_passes=False),
        scratch_types=[
            pltpu.VMEM((CPP, EC), jnp.int32),
            pltpu.VMEM((CPP, EC), jnp.int32),
            pltpu.VMEM((NBUF, EC, D), jnp.float32),
            pltpu.VMEM_SHARED((NP, D), jnp.float32),
            pltpu.SemaphoreType.DMA,
            pltpu.SemaphoreType.DMA,
        ],
    )(_spmm_body)
    return k(hs, src4, dst4, zeros2d)


def _pair_body(h_hbm, pi_hbm, out_hbm, idx_v, rows_v, gsem, wsem):
    cid = lax.axis_index("c")
    sid = lax.axis_index("s")
    wid = cid * NS + sid
    nch = 2 * PCH

    pltpu.sync_copy(pi_hbm.at[wid], idx_v)

    pltpu.async_copy(h_hbm.at[idx_v.at[0]], rows_v.at[0], gsem)

    def step(j, carry):
        b = lax.rem(j, 3)
        pltpu.make_async_copy(h_hbm.at[pl.ds(0, 128)], rows_v.at[b],
                              gsem).wait()
        kk = j // PCH
        c = lax.rem(j, PCH)
        pltpu.async_copy(rows_v.at[b],
                         out_hbm.at[kk, pl.ds(wid * PPW + c * 128, 128)],
                         wsem)

        @pl.when(j >= 2)
        def _():
            pltpu.make_async_copy(rows_v.at[0],
                                  out_hbm.at[0, pl.ds(0, 128)], wsem).wait()

        @pl.when(j + 1 < nch)
        def _():
            b2 = lax.rem(j + 1, 3)
            pltpu.async_copy(h_hbm.at[idx_v.at[j + 1]], rows_v.at[b2], gsem)
        return carry
    lax.fori_loop(0, nch, step, 0)

    pltpu.make_async_copy(rows_v.at[0], out_hbm.at[0, pl.ds(0, 128)],
                          wsem).wait()
    pltpu.make_async_copy(rows_v.at[0], out_hbm.at[0, pl.ds(0, 128)],
                          wsem).wait()


def _pair_call(h, pairs3):
    k = functools.partial(
        pl.kernel,
        out_type=jax.ShapeDtypeStruct((2, PP, D), jnp.float32),
        mesh=plsc.VectorSubcoreMesh(**_MESH),
        compiler_params=pltpu.CompilerParams(needs_layout_passes=False),
        scratch_types=[
            pltpu.VMEM((2 * PCH, 128), jnp.int32),
            pltpu.VMEM((3, 128, D), jnp.float32),
            pltpu.SemaphoreType.DMA,
            pltpu.SemaphoreType.DMA,
        ],
    )(_pair_body)
    return k(h, pairs3)


def _mm_scale_body(x_ref, w_ref, deg_ref, hs_ref, dinv_ref):
    deg = deg_ref[0] + deg_ref[1] + 1.0
    dinv = lax.rsqrt(deg)
    h = jnp.dot(x_ref[...], w_ref[...], preferred_element_type=jnp.float32)
    hs_ref[...] = h * dinv
    dinv_ref[...] = dinv


def _mm_scale_call(x_p, W0, deg3):
    return pl.pallas_call(
        _mm_scale_body,
        out_shape=[jax.ShapeDtypeStruct((NP, D), jnp.float32),
                   jax.ShapeDtypeStruct((NP, 1), jnp.float32)],
    )(x_p, W0, deg3)


def _bn_core(acc_ref, hsp_ref, dinv_ref, b_ref, g_ref, bta_ref):
    t = (acc_ref[0] + acc_ref[1] + hsp_ref[...]) * dinv_ref[...] + b_ref[...]
    rid = lax.broadcasted_iota(jnp.int32, (NP, 1), 0)
    msk = rid < N
    tm = jnp.where(msk, t, 0.0)
    mean = jnp.sum(tm, axis=0, keepdims=True) * (1.0 / N)
    sq = jnp.sum(tm * tm, axis=0, keepdims=True) * (1.0 / N)
    var = sq - mean * mean
    y = (t - mean) * lax.rsqrt(var + 1e-5) * g_ref[...] + bta_ref[...]
    return jnp.maximum(y, 0.0)


def _bn_mm_body(acc_ref, hsp_ref, dinv_ref, b_ref, g_ref, bta_ref, w_ref,
                out_ref):
    y = _bn_core(acc_ref, hsp_ref, dinv_ref, b_ref, g_ref, bta_ref)
    out_ref[...] = jnp.dot(y, w_ref[...],
                           preferred_element_type=jnp.float32) * dinv_ref[...]


def _bn_mm_call(acc, hs_prev, dinv, b, g, bta, W):
    return pl.pallas_call(
        _bn_mm_body,
        out_shape=jax.ShapeDtypeStruct((NP, D), jnp.float32),
    )(acc, hs_prev, dinv, b.reshape(1, D), g.reshape(1, D),
      bta.reshape(1, D), W)


def _bn_body(acc_ref, hsp_ref, dinv_ref, b_ref, g_ref, bta_ref, out_ref):
    out_ref[...] = _bn_core(acc_ref, hsp_ref, dinv_ref, b_ref, g_ref, bta_ref)


def _bn_call(acc, hs_prev, dinv, b, g, bta):
    return pl.pallas_call(
        _bn_body,
        out_shape=jax.ShapeDtypeStruct((NP, D), jnp.float32),
    )(acc, hs_prev, dinv, b.reshape(1, D), g.reshape(1, D), bta.reshape(1, D))


def _mlp_body(e_ref, w1a_ref, w1b_ref, b1_ref, w2_ref, b2_ref, w3_ref, b3_ref,
              out_ref):
    z = (jnp.dot(e_ref[0], w1a_ref[...], preferred_element_type=jnp.float32)
         + jnp.dot(e_ref[1], w1b_ref[...], preferred_element_type=jnp.float32)
         + b1_ref[...])
    z = jnp.maximum(z, 0.0)
    z = jnp.dot(z, w2_ref[...], preferred_element_type=jnp.float32) + b2_ref[...]
    z = jnp.maximum(z, 0.0)
    out_ref[...] = (jnp.dot(z, w3_ref[...], preferred_element_type=jnp.float32)
                    + b3_ref[...])


def _mlp_call(e, Wc1a, Wc1b, bc1, Wc2, bc2, Wc3, bc3):
    H2 = Wc2.shape[1]
    O = Wc3.shape[1]
    return pl.pallas_call(
        _mlp_body,
        out_shape=jax.ShapeDtypeStruct((PP, O), jnp.float32),
    )(e, Wc1a, Wc1b, bc1.reshape(1, D), Wc2, bc2.reshape(1, H2), Wc3,
      bc3.reshape(1, O))


def kernel(x, edge_index, drug_pairs, W0, b0, g0, beta0, W1, b1, g1, beta1,
           Wc1, bc1, Wc2, bc2, Wc3, bc3):
    f32 = jnp.float32
    i32 = jnp.int32
    src = edge_index[0]
    dst = edge_index[1]
    pad_e = EPAD - E
    src_p = jnp.concatenate([src, jnp.zeros((pad_e,), i32)])
    dst_p = jnp.concatenate([dst, jnp.full((pad_e,), N, i32)])
    src4 = src_p.reshape(NW, PH, CPP, EC)
    dst4 = dst_p.reshape(NW, PH, CPP, EC)
    x_p = jnp.concatenate([x, jnp.zeros((NP - N, D), f32)])
    zeros2d = jnp.zeros((NP, D), f32)
    pairs_pad = jnp.concatenate(
        [drug_pairs.T.astype(i32), jnp.zeros((2, PP - P), i32)], axis=1
    ).reshape(2, NW, PCH, 128)
    pairs = jnp.concatenate([pairs_pad[0], pairs_pad[1]], axis=1)

    dego = _deg_call(dst_p)
    deg3 = dego.reshape(NC, NP, 1)
    hs1, dinv = _mm_scale_call(x_p, W0, deg3)
    acc1 = _spmm_call(hs1, src4, dst4, zeros2d)
    hs2 = _bn_mm_call(acc1, hs1, dinv, b0, g0, beta0, W1)
    acc2 = _spmm_call(hs2, src4, dst4, zeros2d)
    hfin = _bn_call(acc2, hs2, dinv, b1, g1, beta1)
    e = _pair_call(hfin, pairs)
    out = _mlp_call(e, Wc1[:D], Wc1[D:], bc1, Wc2, bc2, Wc3, bc3)
    return out[:P]


# speedup vs baseline: 3.3319x; 3.3319x over previous
import functools

import jax
import jax.numpy as jnp
from jax import lax
from jax.experimental import pallas as pl
from jax.experimental.pallas import tpu as pltpu
from jax.experimental.pallas import tpu_sc as plsc

N = 10000
NP = 10240
E = 320000
D = 128
NC = 2
NS = 16
NW = NC * NS
EPW = 10240
EPAD = NW * EPW
EC = 64
PH = 4
CPP = EPW // PH // EC
P = 20000
PP = 20480
PPW = PP // NW
PCH = PPW // 128
RPS = NP // NS

_MESH = dict(core_axis_name="c", subcore_axis_name="s", num_cores=NC,
             num_subcores=NS)


def _deg_body(dst_hbm, out_hbm, idx_v, acc_v, part_v, res_v, stage_sh):
    cid = lax.axis_index("c")
    sid = lax.axis_index("s")
    wid = cid * NS + sid

    def zero(i, c):
        acc_v[pl.ds(i * 16, 16)] = jnp.zeros((16,), jnp.float32)
        return c
    lax.fori_loop(0, NP // 16, zero, 0)

    pltpu.sync_copy(dst_hbm.at[pl.ds(wid * EPW, EPW)], idx_v)

    ones = jnp.ones((16,), jnp.float32)

    def step(i, c):
        iv = idx_v[pl.ds(i * 16, 16)]
        plsc.addupdate_scatter(acc_v, [iv], ones)
        return c
    lax.fori_loop(0, EPW // 16, step, 0)

    pltpu.sync_copy(acc_v, stage_sh.at[sid])
    plsc.subcore_barrier()

    r0 = sid * RPS
    for t in range(NS):
        pltpu.sync_copy(stage_sh.at[t, pl.ds(r0, RPS)], part_v.at[t])

    def red(j, c):
        sl = pl.ds(j * 16, 16)
        s = part_v[0, sl]
        for t in range(1, NS):
            s = s + part_v[t, sl]
        res_v[sl] = s
        return c
    lax.fori_loop(0, RPS // 16, red, 0)

    pltpu.sync_copy(res_v, out_hbm.at[cid, pl.ds(r0, RPS)])


def _deg_call(dst_flat):
    k = functools.partial(
        pl.kernel,
        out_type=jax.ShapeDtypeStruct((NC, NP), jnp.float32),
        mesh=plsc.VectorSubcoreMesh(**_MESH),
        compiler_params=pltpu.CompilerParams(needs_layout_passes=False),
        scratch_types=[
            pltpu.VMEM((EPW,), jnp.int32),
            pltpu.VMEM((NP,), jnp.float32),
            pltpu.VMEM((NS, RPS), jnp.float32),
            pltpu.VMEM((RPS,), jnp.float32),
            pltpu.VMEM_SHARED((NS, NP), jnp.float32),
        ],
    )(_deg_body)
    return k(dst_flat)


NBUF = 4
GDEPTH = 3
SLAG = 1


def _spmm_body(hs_hbm, si_hbm, di_hbm, z_hbm, out_hbm,
               sidx_v, didx_v, rows_v, acc_sh, gsem, ssem):
    cid = lax.axis_index("c")
    sid = lax.axis_index("s")
    wid = cid * NS + sid
    r0 = sid * RPS

    pltpu.sync_copy(z_hbm.at[pl.ds(r0, RPS)], acc_sh.at[pl.ds(r0, RPS)])
    plsc.subcore_barrier()

    def phase(p, carry0):
        pltpu.sync_copy(si_hbm.at[wid, p], sidx_v)
        pltpu.sync_copy(di_hbm.at[wid, p], didx_v)

        for b in range(GDEPTH):
            pltpu.async_copy(hs_hbm.at[sidx_v.at[b]], rows_v.at[b], gsem)

        def step(i, carry):
            b = lax.rem(i, NBUF)
            pltpu.make_async_copy(hs_hbm.at[pl.ds(0, EC)], rows_v.at[b],
                                  gsem).wait()
            pltpu.async_copy(rows_v.at[b], acc_sh.at[didx_v.at[i]], ssem,
                             add=True)

            @pl.when(i >= SLAG)
            def _():
                pltpu.make_async_copy(rows_v.at[0], acc_sh.at[pl.ds(0, EC)],
                                      ssem).wait()

            @pl.when(i + GDEPTH < CPP)
            def _():
                b2 = lax.rem(i + GDEPTH, NBUF)
                pltpu.async_copy(hs_hbm.at[sidx_v.at[i + GDEPTH]],
                                 rows_v.at[b2], gsem)
            return carry
        lax.fori_loop(0, CPP, step, 0)

        for _ in range(SLAG):
            pltpu.make_async_copy(rows_v.at[0], acc_sh.at[pl.ds(0, EC)],
                                  ssem).wait()
        return carry0
    lax.fori_loop(0, PH, phase, 0)

    plsc.subcore_barrier()
    pltpu.sync_copy(acc_sh.at[pl.ds(r0, RPS)], out_hbm.at[cid, pl.ds(r0, RPS)])


def _spmm_call(hs, src4, dst4, zeros2d):
    k = functools.partial(
        pl.kernel,
        out_type=jax.ShapeDtypeStruct((NC, NP, D), jnp.float32),
        mesh=plsc.VectorSubcoreMesh(**_MESH),
        compiler_params=pltpu.CompilerParams(needs_layout_passes=False),
        scratch_types=[
            pltpu.VMEM((CPP, EC), jnp.int32),
            pltpu.VMEM((CPP, EC), jnp.int32),
            pltpu.VMEM((NBUF, EC, D), jnp.float32),
            pltpu.VMEM_SHARED((NP, D), jnp.float32),
            pltpu.SemaphoreType.DMA,
            pltpu.SemaphoreType.DMA,
        ],
    )(_spmm_body)
    return k(hs, src4, dst4, zeros2d)


def _pair_body(h_hbm, pi_hbm, out_hbm, idx_v, rows_v, gsem, wsem):
    cid = lax.axis_index("c")
    sid = lax.axis_index("s")
    wid = cid * NS + sid
    nch = 2 * PCH

    pltpu.sync_copy(pi_hbm.at[wid], idx_v)

    pltpu.async_copy(h_hbm.at[idx_v.at[0]], rows_v.at[0], gsem)

    def step(j, carry):
        b = lax.rem(j, 3)
        pltpu.make_async_copy(h_hbm.at[pl.ds(0, 128)], rows_v.at[b],
                              gsem).wait()
        kk = j // PCH
        c = lax.rem(j, PCH)
        pltpu.async_copy(rows_v.at[b],
                         out_hbm.at[kk, pl.ds(wid * PPW + c * 128, 128)],
                         wsem)

        @pl.when(j >= 2)
        def _():
            pltpu.make_async_copy(rows_v.at[0],
                                  out_hbm.at[0, pl.ds(0, 128)], wsem).wait()

        @pl.when(j + 1 < nch)
        def _():
            b2 = lax.rem(j + 1, 3)
            pltpu.async_copy(h_hbm.at[idx_v.at[j + 1]], rows_v.at[b2], gsem)
        return carry
    lax.fori_loop(0, nch, step, 0)

    pltpu.make_async_copy(rows_v.at[0], out_hbm.at[0, pl.ds(0, 128)],
                          wsem).wait()
    pltpu.make_async_copy(rows_v.at[0], out_hbm.at[0, pl.ds(0, 128)],
                          wsem).wait()


def _pair_call(h, pairs3):
    k = functools.partial(
        pl.kernel,
        out_type=jax.ShapeDtypeStruct((2, PP, D), jnp.float32),
        mesh=plsc.VectorSubcoreMesh(**_MESH),
        compiler_params=pltpu.CompilerParams(needs_layout_passes=False),
        scratch_types=[
            pltpu.VMEM((2 * PCH, 128), jnp.int32),
            pltpu.VMEM((3, 128, D), jnp.float32),
            pltpu.SemaphoreType.DMA,
            pltpu.SemaphoreType.DMA,
        ],
    )(_pair_body)
    return k(h, pairs3)


def _mm_scale_body(x_ref, w_ref, deg_ref, hs_ref, dinv_ref):
    deg = deg_ref[0] + deg_ref[1] + 1.0
    dinv = lax.rsqrt(deg)
    h = jnp.dot(x_ref[...], w_ref[...], preferred_element_type=jnp.float32)
    hs_ref[...] = h * dinv
    dinv_ref[...] = dinv


def _mm_scale_call(x_p, W0, deg3):
    return pl.pallas_call(
        _mm_scale_body,
        out_shape=[jax.ShapeDtypeStruct((NP, D), jnp.float32),
                   jax.ShapeDtypeStruct((NP, 1), jnp.float32)],
    )(x_p, W0, deg3)


def _bn_core(acc_ref, hsp_ref, dinv_ref, b_ref, g_ref, bta_ref):
    t = (acc_ref[0] + acc_ref[1] + hsp_ref[...]) * dinv_ref[...] + b_ref[...]
    rid = lax.broadcasted_iota(jnp.int32, (NP, 1), 0)
    msk = rid < N
    tm = jnp.where(msk, t, 0.0)
    mean = jnp.sum(tm, axis=0, keepdims=True) * (1.0 / N)
    sq = jnp.sum(tm * tm, axis=0, keepdims=True) * (1.0 / N)
    var = sq - mean * mean
    y = (t - mean) * lax.rsqrt(var + 1e-5) * g_ref[...] + bta_ref[...]
    return jnp.maximum(y, 0.0)


def _bn_mm_body(acc_ref, hsp_ref, dinv_ref, b_ref, g_ref, bta_ref, w_ref,
                out_ref):
    y = _bn_core(acc_ref, hsp_ref, dinv_ref, b_ref, g_ref, bta_ref)
    out_ref[...] = jnp.dot(y, w_ref[...],
                           preferred_element_type=jnp.float32) * dinv_ref[...]


def _bn_mm_call(acc, hs_prev, dinv, b, g, bta, W):
    return pl.pallas_call(
        _bn_mm_body,
        out_shape=jax.ShapeDtypeStruct((NP, D), jnp.float32),
    )(acc, hs_prev, dinv, b.reshape(1, D), g.reshape(1, D),
      bta.reshape(1, D), W)


def _bn_body(acc_ref, hsp_ref, dinv_ref, b_ref, g_ref, bta_ref, out_ref):
    out_ref[...] = _bn_core(acc_ref, hsp_ref, dinv_ref, b_ref, g_ref, bta_ref)


def _bn_call(acc, hs_prev, dinv, b, g, bta):
    return pl.pallas_call(
        _bn_body,
        out_shape=jax.ShapeDtypeStruct((NP, D), jnp.float32),
    )(acc, hs_prev, dinv, b.reshape(1, D), g.reshape(1, D), bta.reshape(1, D))


def _mlp_body(e_ref, w1a_ref, w1b_ref, b1_ref, w2_ref, b2_ref, w3_ref, b3_ref,
              out_ref):
    z = (jnp.dot(e_ref[0], w1a_ref[...], preferred_element_type=jnp.float32)
         + jnp.dot(e_ref[1], w1b_ref[...], preferred_element_type=jnp.float32)
         + b1_ref[...])
    z = jnp.maximum(z, 0.0)
    z = jnp.dot(z, w2_ref[...], preferred_element_type=jnp.float32) + b2_ref[...]
    z = jnp.maximum(z, 0.0)
    out_ref[...] = (jnp.dot(z, w3_ref[...], preferred_element_type=jnp.float32)
                    + b3_ref[...])


def _mlp_call(e, Wc1a, Wc1b, bc1, Wc2, bc2, Wc3, bc3):
    H2 = Wc2.shape[1]
    O = Wc3.shape[1]
    return pl.pallas_call(
        _mlp_body,
        out_shape=jax.ShapeDtypeStruct((PP, O), jnp.float32),
    )(e, Wc1a, Wc1b, bc1.reshape(1, D), Wc2, bc2.reshape(1, H2), Wc3,
      bc3.reshape(1, O))


def kernel(x, edge_index, drug_pairs, W0, b0, g0, beta0, W1, b1, g1, beta1,
           Wc1, bc1, Wc2, bc2, Wc3, bc3):
    f32 = jnp.float32
    i32 = jnp.int32
    src = edge_index[0]
    dst = edge_index[1]
    pad_e = EPAD - E
    pad_r = N + jnp.arange(pad_e, dtype=i32) % (NP - N)
    src_p = jnp.concatenate([src, pad_r])
    dst_p = jnp.concatenate([dst, pad_r])
    src4 = src_p.reshape(NW, PH, CPP, EC)
    dst4 = dst_p.reshape(NW, PH, CPP, EC)
    x_p = jnp.concatenate([x, jnp.zeros((NP - N, D), f32)])
    zeros2d = jnp.zeros((NP, D), f32)
    pad_p = jnp.arange(PP - P, dtype=i32) % N
    pairs_pad = jnp.concatenate(
        [drug_pairs.T.astype(i32), jnp.stack([pad_p, pad_p])], axis=1
    ).reshape(2, NW, PCH, 128)
    pairs = jnp.concatenate([pairs_pad[0], pairs_pad[1]], axis=1)

    dego = _deg_call(dst_p)
    deg3 = dego.reshape(NC, NP, 1)
    hs1, dinv = _mm_scale_call(x_p, W0, deg3)
    acc1 = _spmm_call(hs1, src4, dst4, zeros2d)
    hs2 = _bn_mm_call(acc1, hs1, dinv, b0, g0, beta0, W1)
    acc2 = _spmm_call(hs2, src4, dst4, zeros2d)
    hfin = _bn_call(acc2, hs2, dinv, b1, g1, beta1)
    e = _pair_call(hfin, pairs)
    out = _mlp_call(e, Wc1[:D], Wc1[D:], bc1, Wc2, bc2, Wc3, bc3)
    return out[:P]


# speedup vs baseline: 3.5546x; 1.0668x over previous
import functools

import jax
import jax.numpy as jnp
from jax import lax
from jax.experimental import pallas as pl
from jax.experimental.pallas import tpu as pltpu
from jax.experimental.pallas import tpu_sc as plsc

N = 10000
NP = 10240
E = 320000
D = 128
NC = 2
NS = 16
NW = NC * NS
EPW = 10240
EPAD = NW * EPW
EC = 64
PH = 4
CPP = EPW // PH // EC
P = 20000
PP = 20480
PPW = PP // NW
PCH = PPW // 128
RPS = NP // NS

_MESH = dict(core_axis_name="c", subcore_axis_name="s", num_cores=NC,
             num_subcores=NS)


EPP = EPW // PH


def _deg_body(ei_hbm, out_hbm, idx_v, acc_v, part_v, res_v, stage_sh):
    cid = lax.axis_index("c")
    sid = lax.axis_index("s")
    wid = cid * NS + sid

    def zero(i, c):
        acc_v[pl.ds(i * 16, 16)] = jnp.zeros((16,), jnp.float32)
        return c
    lax.fori_loop(0, NP // 16, zero, 0)

    ones = jnp.ones((16,), jnp.float32)

    for ph in range(PH):
        @pl.when((wid < NW - 1) | (ph == 0))
        def _():
            pltpu.sync_copy(ei_hbm.at[1, pl.ds(wid * EPW + ph * EPP, EPP)],
                            idx_v)

            def step(i, c):
                iv = idx_v[pl.ds(i * 16, 16)]
                plsc.addupdate_scatter(acc_v, [iv], ones)
                return c
            lax.fori_loop(0, EPP // 16, step, 0)

    pltpu.sync_copy(acc_v, stage_sh.at[sid])
    plsc.subcore_barrier()

    r0 = sid * RPS
    for t in range(NS):
        pltpu.sync_copy(stage_sh.at[t, pl.ds(r0, RPS)], part_v.at[t])

    def red(j, c):
        sl = pl.ds(j * 16, 16)
        s = part_v[0, sl]
        for t in range(1, NS):
            s = s + part_v[t, sl]
        res_v[sl] = s
        return c
    lax.fori_loop(0, RPS // 16, red, 0)

    pltpu.sync_copy(res_v, out_hbm.at[cid, pl.ds(r0, RPS)])


def _deg_call(edge_index):
    k = functools.partial(
        pl.kernel,
        out_type=jax.ShapeDtypeStruct((NC, NP), jnp.float32),
        mesh=plsc.VectorSubcoreMesh(**_MESH),
        compiler_params=pltpu.CompilerParams(needs_layout_passes=False),
        scratch_types=[
            pltpu.VMEM((EPP,), jnp.int32),
            pltpu.VMEM((NP,), jnp.float32),
            pltpu.VMEM((NS, RPS), jnp.float32),
            pltpu.VMEM((RPS,), jnp.float32),
            pltpu.VMEM_SHARED((NS, NP), jnp.float32),
        ],
    )(_deg_body)
    return k(edge_index)


NBUF = 4
GDEPTH = 3
SLAG = 1


def _spmm_body(hs_hbm, ei_hbm, z_hbm, out_hbm,
               sidx_v, didx_v, rows_v, acc_sh, gsem, ssem):
    cid = lax.axis_index("c")
    sid = lax.axis_index("s")
    wid = cid * NS + sid
    r0 = sid * RPS

    pltpu.sync_copy(z_hbm.at[pl.ds(r0, RPS)], acc_sh.at[pl.ds(r0, RPS)])
    plsc.subcore_barrier()

    for p in range(PH):
        @pl.when((wid < NW - 1) | (p == 0))
        def _():
            e0 = wid * EPW + p * EPP
            pltpu.sync_copy(ei_hbm.at[0, pl.ds(e0, EPP)], sidx_v)
            pltpu.sync_copy(ei_hbm.at[1, pl.ds(e0, EPP)], didx_v)

            for b in range(GDEPTH):
                pltpu.async_copy(hs_hbm.at[sidx_v.at[pl.ds(b * EC, EC)]],
                                 rows_v.at[b], gsem)

            def step(i, carry):
                b = lax.rem(i, NBUF)
                pltpu.make_async_copy(hs_hbm.at[pl.ds(0, EC)], rows_v.at[b],
                                      gsem).wait()
                pltpu.async_copy(rows_v.at[b],
                                 acc_sh.at[didx_v.at[pl.ds(i * EC, EC)]],
                                 ssem, add=True)

                @pl.when(i >= SLAG)
                def _():
                    pltpu.make_async_copy(rows_v.at[0],
                                          acc_sh.at[pl.ds(0, EC)],
                                          ssem).wait()

                @pl.when(i + GDEPTH < CPP)
                def _():
                    b2 = lax.rem(i + GDEPTH, NBUF)
                    pltpu.async_copy(
                        hs_hbm.at[sidx_v.at[pl.ds((i + GDEPTH) * EC, EC)]],
                        rows_v.at[b2], gsem)
                return carry
            lax.fori_loop(0, CPP, step, 0)

            for _ in range(SLAG):
                pltpu.make_async_copy(rows_v.at[0], acc_sh.at[pl.ds(0, EC)],
                                      ssem).wait()

    plsc.subcore_barrier()
    pltpu.sync_copy(acc_sh.at[pl.ds(r0, RPS)], out_hbm.at[cid, pl.ds(r0, RPS)])


def _spmm_call(hs, edge_index, zeros2d):
    k = functools.partial(
        pl.kernel,
        out_type=jax.ShapeDtypeStruct((NC, NP, D), jnp.float32),
        mesh=plsc.VectorSubcoreMesh(**_MESH),
        compiler_params=pltpu.CompilerParams(needs_layout_passes=False),
        scratch_types=[
            pltpu.VMEM((EPP,), jnp.int32),
            pltpu.VMEM((EPP,), jnp.int32),
            pltpu.VMEM((NBUF, EC, D), jnp.float32),
            pltpu.VMEM_SHARED((NP, D), jnp.float32),
            pltpu.SemaphoreType.DMA,
            pltpu.SemaphoreType.DMA,
        ],
    )(_spmm_body)
    return k(hs, edge_index, zeros2d)


def _pair_body(h_hbm, pi_hbm, out_hbm, idx_v, rows_v, gsem, wsem):
    cid = lax.axis_index("c")
    sid = lax.axis_index("s")
    wid = cid * NS + sid
    nch = 2 * PCH

    pltpu.sync_copy(pi_hbm.at[wid], idx_v)

    pltpu.async_copy(h_hbm.at[idx_v.at[0]], rows_v.at[0], gsem)

    def step(j, carry):
        b = lax.rem(j, 3)
        pltpu.make_async_copy(h_hbm.at[pl.ds(0, 128)], rows_v.at[b],
                              gsem).wait()
        kk = j // PCH
        c = lax.rem(j, PCH)
        pltpu.async_copy(rows_v.at[b],
                         out_hbm.at[kk, pl.ds(wid * PPW + c * 128, 128)],
                         wsem)

        @pl.when(j >= 2)
        def _():
            pltpu.make_async_copy(rows_v.at[0],
                                  out_hbm.at[0, pl.ds(0, 128)], wsem).wait()

        @pl.when(j + 1 < nch)
        def _():
            b2 = lax.rem(j + 1, 3)
            pltpu.async_copy(h_hbm.at[idx_v.at[j + 1]], rows_v.at[b2], gsem)
        return carry
    lax.fori_loop(0, nch, step, 0)

    pltpu.make_async_copy(rows_v.at[0], out_hbm.at[0, pl.ds(0, 128)],
                          wsem).wait()
    pltpu.make_async_copy(rows_v.at[0], out_hbm.at[0, pl.ds(0, 128)],
                          wsem).wait()


def _pair_call(h, pairs3):
    k = functools.partial(
        pl.kernel,
        out_type=jax.ShapeDtypeStruct((2, PP, D), jnp.float32),
        mesh=plsc.VectorSubcoreMesh(**_MESH),
        compiler_params=pltpu.CompilerParams(needs_layout_passes=False),
        scratch_types=[
            pltpu.VMEM((2 * PCH, 128), jnp.int32),
            pltpu.VMEM((3, 128, D), jnp.float32),
            pltpu.SemaphoreType.DMA,
            pltpu.SemaphoreType.DMA,
        ],
    )(_pair_body)
    return k(h, pairs3)


def _mm0_body(x_ref, w_ref, h0_ref):
    h0_ref[...] = jnp.dot(x_ref[...], w_ref[...],
                          preferred_element_type=jnp.float32)


def _mm0_call(x, W0):
    return pl.pallas_call(
        _mm0_body,
        out_shape=jax.ShapeDtypeStruct((N, D), jnp.float32),
    )(x, W0)


def _scale_body(h0_ref, deg_ref, hs_ref, dinv_ref):
    deg = deg_ref[0] + deg_ref[1] + 1.0
    dinv = lax.rsqrt(deg)
    hs_ref[pl.ds(0, N)] = h0_ref[...] * dinv[:N]
    hs_ref[pl.ds(N, NP - N)] = jnp.zeros((NP - N, D), jnp.float32)
    dinv_ref[...] = dinv


def _scale_call(h0, deg3):
    return pl.pallas_call(
        _scale_body,
        out_shape=[jax.ShapeDtypeStruct((NP, D), jnp.float32),
                   jax.ShapeDtypeStruct((NP, 1), jnp.float32)],
    )(h0, deg3)


def _bn_core(acc_ref, hsp_ref, dinv_ref, b_ref, g_ref, bta_ref):
    t = (acc_ref[0] + acc_ref[1] + hsp_ref[...]) * dinv_ref[...] + b_ref[...]
    rid = lax.broadcasted_iota(jnp.int32, (NP, 1), 0)
    msk = rid < N
    tm = jnp.where(msk, t, 0.0)
    mean = jnp.sum(tm, axis=0, keepdims=True) * (1.0 / N)
    sq = jnp.sum(tm * tm, axis=0, keepdims=True) * (1.0 / N)
    var = sq - mean * mean
    y = (t - mean) * lax.rsqrt(var + 1e-5) * g_ref[...] + bta_ref[...]
    return jnp.maximum(y, 0.0)


def _bn_mm_body(acc_ref, hsp_ref, dinv_ref, b_ref, g_ref, bta_ref, w_ref,
                out_ref):
    y = _bn_core(acc_ref, hsp_ref, dinv_ref, b_ref, g_ref, bta_ref)
    out_ref[...] = jnp.dot(y, w_ref[...],
                           preferred_element_type=jnp.float32) * dinv_ref[...]


def _bn_mm_call(acc, hs_prev, dinv, b, g, bta, W):
    return pl.pallas_call(
        _bn_mm_body,
        out_shape=jax.ShapeDtypeStruct((NP, D), jnp.float32),
    )(acc, hs_prev, dinv, b.reshape(1, D), g.reshape(1, D),
      bta.reshape(1, D), W)


def _bn_body(acc_ref, hsp_ref, dinv_ref, b_ref, g_ref, bta_ref, out_ref):
    out_ref[...] = _bn_core(acc_ref, hsp_ref, dinv_ref, b_ref, g_ref, bta_ref)


def _bn_call(acc, hs_prev, dinv, b, g, bta):
    return pl.pallas_call(
        _bn_body,
        out_shape=jax.ShapeDtypeStruct((NP, D), jnp.float32),
    )(acc, hs_prev, dinv, b.reshape(1, D), g.reshape(1, D), bta.reshape(1, D))


def _mlp_body(e_ref, w1a_ref, w1b_ref, b1_ref, w2_ref, b2_ref, w3_ref, b3_ref,
              out_ref):
    bf = jnp.bfloat16
    z = (jnp.dot(e_ref[0].astype(bf), w1a_ref[...].astype(bf),
                 preferred_element_type=jnp.float32)
         + jnp.dot(e_ref[1].astype(bf), w1b_ref[...].astype(bf),
                   preferred_element_type=jnp.float32)
         + b1_ref[...])
    z = jnp.maximum(z, 0.0).astype(bf)
    z = jnp.dot(z, w2_ref[...].astype(bf),
                preferred_element_type=jnp.float32) + b2_ref[...]
    z = jnp.maximum(z, 0.0).astype(bf)
    z = (jnp.dot(z, w3_ref[...].astype(bf), preferred_element_type=jnp.float32)
         + b3_ref[...])
    out_ref[...] = z[:P]


def _mlp_call(e, Wc1a, Wc1b, bc1, Wc2, bc2, Wc3, bc3):
    H2 = Wc2.shape[1]
    O = Wc3.shape[1]
    return pl.pallas_call(
        _mlp_body,
        out_shape=jax.ShapeDtypeStruct((P, O), jnp.float32),
    )(e, Wc1a, Wc1b, bc1.reshape(1, D), Wc2, bc2.reshape(1, H2), Wc3,
      bc3.reshape(1, O))


def kernel(x, edge_index, drug_pairs, W0, b0, g0, beta0, W1, b1, g1, beta1,
           Wc1, bc1, Wc2, bc2, Wc3, bc3):
    f32 = jnp.float32
    i32 = jnp.int32
    zeros2d = jnp.zeros((NP, D), f32)
    pad_p = jnp.arange(PP - P, dtype=i32) % N
    pairs_pad = jnp.concatenate(
        [drug_pairs.T.astype(i32), jnp.stack([pad_p, pad_p])], axis=1
    ).reshape(2, NW, PCH, 128)
    pairs = jnp.concatenate([pairs_pad[0], pairs_pad[1]], axis=1)

    dego = _deg_call(edge_index)
    deg3 = dego.reshape(NC, NP, 1)
    h0 = _mm0_call(x, W0)
    hs1, dinv = _scale_call(h0, deg3)
    acc1 = _spmm_call(hs1, edge_index, zeros2d)
    hs2 = _bn_mm_call(acc1, hs1, dinv, b0, g0, beta0, W1)
    acc2 = _spmm_call(hs2, edge_index, zeros2d)
    hfin = _bn_call(acc2, hs2, dinv, b1, g1, beta1)
    e = _pair_call(hfin, pairs)
    return _mlp_call(e, Wc1[:D], Wc1[D:], bc1, Wc2, bc2, Wc3, bc3)


# speedup vs baseline: 3.7144x; 1.0450x over previous
import functools

import jax
import jax.numpy as jnp
from jax import lax
from jax.experimental import pallas as pl
from jax.experimental.pallas import tpu as pltpu
from jax.experimental.pallas import tpu_sc as plsc

N = 10000
NP = 10240
E = 320000
D = 128
NC = 2
NS = 16
NW = NC * NS
EPW = 10240
EPAD = NW * EPW
EC = 64
PH = 4
CPP = EPW // PH // EC
P = 20000
PP = 20480
PPW = PP // NW
PCH = PPW // 128
RPS = NP // NS

_MESH = dict(core_axis_name="c", subcore_axis_name="s", num_cores=NC,
             num_subcores=NS)


EPP = EPW // PH


def _deg_body(ei_hbm, out_hbm, idx_v, acc_v, part_v, res_v, stage_sh):
    cid = lax.axis_index("c")
    sid = lax.axis_index("s")
    wid = cid * NS + sid

    def zero(i, c):
        acc_v[pl.ds(i * 16, 16)] = jnp.zeros((16,), jnp.float32)
        return c
    lax.fori_loop(0, NP // 16, zero, 0)

    ones = jnp.ones((16,), jnp.float32)

    for ph in range(PH):
        @pl.when((wid < NW - 1) | (ph == 0))
        def _():
            pltpu.sync_copy(ei_hbm.at[1, pl.ds(wid * EPW + ph * EPP, EPP)],
                            idx_v)

            def step(i, c):
                iv = idx_v[pl.ds(i * 16, 16)]
                plsc.addupdate_scatter(acc_v, [iv], ones)
                return c
            lax.fori_loop(0, EPP // 16, step, 0)

    pltpu.sync_copy(acc_v, stage_sh.at[sid])
    plsc.subcore_barrier()

    r0 = sid * RPS
    for t in range(NS):
        pltpu.sync_copy(stage_sh.at[t, pl.ds(r0, RPS)], part_v.at[t])

    def red(j, c):
        sl = pl.ds(j * 16, 16)
        s = part_v[0, sl]
        for t in range(1, NS):
            s = s + part_v[t, sl]
        res_v[sl] = s
        return c
    lax.fori_loop(0, RPS // 16, red, 0)

    pltpu.sync_copy(res_v, out_hbm.at[cid, pl.ds(r0, RPS)])


def _deg_call(edge_index):
    k = functools.partial(
        pl.kernel,
        out_type=jax.ShapeDtypeStruct((NC, NP), jnp.float32),
        mesh=plsc.VectorSubcoreMesh(**_MESH),
        compiler_params=pltpu.CompilerParams(needs_layout_passes=False),
        scratch_types=[
            pltpu.VMEM((EPP,), jnp.int32),
            pltpu.VMEM((NP,), jnp.float32),
            pltpu.VMEM((NS, RPS), jnp.float32),
            pltpu.VMEM((RPS,), jnp.float32),
            pltpu.VMEM_SHARED((NS, NP), jnp.float32),
        ],
    )(_deg_body)
    return k(edge_index)


NBUF = 4
GDEPTH = 3
SLAG = 1


def _spmm_body(hs_hbm, ei_hbm, z_hbm, out_hbm,
               sidx_v, didx_v, rows_v, acc_sh, gsem, ssem):
    cid = lax.axis_index("c")
    sid = lax.axis_index("s")
    wid = cid * NS + sid
    r0 = sid * RPS

    pltpu.sync_copy(z_hbm, acc_sh.at[pl.ds(r0, RPS)])
    plsc.subcore_barrier()

    for p in range(PH):
        @pl.when((wid < NW - 1) | (p == 0))
        def _():
            e0 = wid * EPW + p * EPP
            pltpu.sync_copy(ei_hbm.at[0, pl.ds(e0, EPP)], sidx_v)
            pltpu.sync_copy(ei_hbm.at[1, pl.ds(e0, EPP)], didx_v)

            for b in range(GDEPTH):
                pltpu.async_copy(hs_hbm.at[sidx_v.at[pl.ds(b * EC, EC)]],
                                 rows_v.at[b], gsem)

            def step(i, carry):
                b = lax.rem(i, NBUF)
                pltpu.make_async_copy(hs_hbm.at[pl.ds(0, EC)], rows_v.at[b],
                                      gsem).wait()
                pltpu.async_copy(rows_v.at[b],
                                 acc_sh.at[didx_v.at[pl.ds(i * EC, EC)]],
                                 ssem, add=True)

                @pl.when(i >= SLAG)
                def _():
                    pltpu.make_async_copy(rows_v.at[0],
                                          acc_sh.at[pl.ds(0, EC)],
                                          ssem).wait()

                @pl.when(i + GDEPTH < CPP)
                def _():
                    b2 = lax.rem(i + GDEPTH, NBUF)
                    pltpu.async_copy(
                        hs_hbm.at[sidx_v.at[pl.ds((i + GDEPTH) * EC, EC)]],
                        rows_v.at[b2], gsem)
                return carry
            lax.fori_loop(0, CPP, step, 0)

            for _ in range(SLAG):
                pltpu.make_async_copy(rows_v.at[0], acc_sh.at[pl.ds(0, EC)],
                                      ssem).wait()

    plsc.subcore_barrier()
    pltpu.sync_copy(acc_sh.at[pl.ds(r0, RPS)], out_hbm.at[cid, pl.ds(r0, RPS)])


def _spmm_call(hs, edge_index, zeros2d):
    k = functools.partial(
        pl.kernel,
        out_type=jax.ShapeDtypeStruct((NC, NP, D), jnp.float32),
        mesh=plsc.VectorSubcoreMesh(**_MESH),
        compiler_params=pltpu.CompilerParams(needs_layout_passes=False),
        scratch_types=[
            pltpu.VMEM((EPP,), jnp.int32),
            pltpu.VMEM((EPP,), jnp.int32),
            pltpu.VMEM((NBUF, EC, D), jnp.float32),
            pltpu.VMEM_SHARED((NP, D), jnp.float32),
            pltpu.SemaphoreType.DMA,
            pltpu.SemaphoreType.DMA,
        ],
    )(_spmm_body)
    return k(hs, edge_index, zeros2d)


def _pair_body(h_hbm, pi_hbm, out_hbm, idx_v, rows_v, gsem, wsem):
    cid = lax.axis_index("c")
    sid = lax.axis_index("s")
    wid = cid * NS + sid
    nch = 2 * PCH

    pltpu.sync_copy(pi_hbm.at[wid], idx_v)

    pltpu.async_copy(h_hbm.at[idx_v.at[0]], rows_v.at[0], gsem)

    def step(j, carry):
        b = lax.rem(j, 3)
        pltpu.make_async_copy(h_hbm.at[pl.ds(0, 128)], rows_v.at[b],
                              gsem).wait()
        kk = j // PCH
        c = lax.rem(j, PCH)
        pltpu.async_copy(rows_v.at[b],
                         out_hbm.at[kk, pl.ds(wid * PPW + c * 128, 128)],
                         wsem)

        @pl.when(j >= 2)
        def _():
            pltpu.make_async_copy(rows_v.at[0],
                                  out_hbm.at[0, pl.ds(0, 128)], wsem).wait()

        @pl.when(j + 1 < nch)
        def _():
            b2 = lax.rem(j + 1, 3)
            pltpu.async_copy(h_hbm.at[idx_v.at[j + 1]], rows_v.at[b2], gsem)
        return carry
    lax.fori_loop(0, nch, step, 0)

    pltpu.make_async_copy(rows_v.at[0], out_hbm.at[0, pl.ds(0, 128)],
                          wsem).wait()
    pltpu.make_async_copy(rows_v.at[0], out_hbm.at[0, pl.ds(0, 128)],
                          wsem).wait()


def _pair_call(h, pairs3):
    k = functools.partial(
        pl.kernel,
        out_type=jax.ShapeDtypeStruct((2, PP, D), jnp.float32),
        mesh=plsc.VectorSubcoreMesh(**_MESH),
        compiler_params=pltpu.CompilerParams(needs_layout_passes=False),
        scratch_types=[
            pltpu.VMEM((2 * PCH, 128), jnp.int32),
            pltpu.VMEM((3, 128, D), jnp.float32),
            pltpu.SemaphoreType.DMA,
            pltpu.SemaphoreType.DMA,
        ],
    )(_pair_body)
    return k(h, pairs3)


def _mm0_body(x_ref, w_ref, h0_ref):
    h0_ref[...] = jnp.dot(x_ref[...], w_ref[...],
                          preferred_element_type=jnp.float32)


def _mm0_call(x, W0):
    return pl.pallas_call(
        _mm0_body,
        out_shape=jax.ShapeDtypeStruct((N, D), jnp.float32),
    )(x, W0)


def _scale_body(h0_ref, deg_ref, hs_ref, dinv_ref):
    deg = deg_ref[0] + deg_ref[1] + 1.0
    dinv = lax.rsqrt(deg).reshape(NP, 1)
    hs_ref[pl.ds(0, N)] = h0_ref[...] * dinv[:N]
    hs_ref[pl.ds(N, NP - N)] = jnp.zeros((NP - N, D), jnp.float32)
    dinv_ref[...] = dinv


def _scale_call(h0, dego):
    return pl.pallas_call(
        _scale_body,
        out_shape=[jax.ShapeDtypeStruct((NP, D), jnp.float32),
                   jax.ShapeDtypeStruct((NP, 1), jnp.float32)],
    )(h0, dego)


def _bn_core(acc_ref, hsp_ref, dinv_ref, b_ref, g_ref, bta_ref):
    t = (acc_ref[0] + acc_ref[1] + hsp_ref[...]) * dinv_ref[...] + b_ref[...]
    rid = lax.broadcasted_iota(jnp.int32, (NP, 1), 0)
    msk = rid < N
    tm = jnp.where(msk, t, 0.0)
    mean = jnp.sum(tm, axis=0, keepdims=True) * (1.0 / N)
    sq = jnp.sum(tm * tm, axis=0, keepdims=True) * (1.0 / N)
    var = sq - mean * mean
    y = (t - mean) * lax.rsqrt(var + 1e-5) * g_ref[...] + bta_ref[...]
    return jnp.maximum(y, 0.0)


def _bn_mm_body(acc_ref, hsp_ref, dinv_ref, b_ref, g_ref, bta_ref, w_ref,
                out_ref):
    y = _bn_core(acc_ref, hsp_ref, dinv_ref, b_ref, g_ref, bta_ref)
    out_ref[...] = jnp.dot(y, w_ref[...],
                           preferred_element_type=jnp.float32) * dinv_ref[...]


def _bn_mm_call(acc, hs_prev, dinv, b, g, bta, W):
    return pl.pallas_call(
        _bn_mm_body,
        out_shape=jax.ShapeDtypeStruct((NP, D), jnp.float32),
    )(acc, hs_prev, dinv, b.reshape(1, D), g.reshape(1, D),
      bta.reshape(1, D), W)


def _bn_body(acc_ref, hsp_ref, dinv_ref, b_ref, g_ref, bta_ref, out_ref):
    out_ref[...] = _bn_core(acc_ref, hsp_ref, dinv_ref, b_ref, g_ref, bta_ref)


def _bn_call(acc, hs_prev, dinv, b, g, bta):
    return pl.pallas_call(
        _bn_body,
        out_shape=jax.ShapeDtypeStruct((NP, D), jnp.float32),
    )(acc, hs_prev, dinv, b.reshape(1, D), g.reshape(1, D), bta.reshape(1, D))


def _mlp_body(e_ref, w1a_ref, w1b_ref, b1_ref, w2_ref, b2_ref, w3_ref, b3_ref,
              out_ref):
    bf = jnp.bfloat16
    z = (jnp.dot(e_ref[0].astype(bf), w1a_ref[...].astype(bf),
                 preferred_element_type=jnp.float32)
         + jnp.dot(e_ref[1].astype(bf), w1b_ref[...].astype(bf),
                   preferred_element_type=jnp.float32)
         + b1_ref[...])
    z = jnp.maximum(z, 0.0).astype(bf)
    z = jnp.dot(z, w2_ref[...].astype(bf),
                preferred_element_type=jnp.float32) + b2_ref[...]
    z = jnp.maximum(z, 0.0).astype(bf)
    z = (jnp.dot(z, w3_ref[...].astype(bf), preferred_element_type=jnp.float32)
         + b3_ref[...])
    out_ref[...] = z


MB = PP // 4


def _mlp_call(e, Wc1a, Wc1b, bc1, Wc2, bc2, Wc3, bc3):
    H2 = Wc2.shape[1]
    O = Wc3.shape[1]
    full = lambda *s: pl.BlockSpec(s, lambda i: (0,) * len(s))
    return pl.pallas_call(
        _mlp_body,
        grid=(4,),
        in_specs=[
            pl.BlockSpec((2, MB, D), lambda i: (0, i, 0)),
            full(D, D), full(D, D), full(1, D),
            full(D, H2), full(1, H2), full(H2, O), full(1, O),
        ],
        out_specs=pl.BlockSpec((MB, O), lambda i: (i, 0)),
        out_shape=jax.ShapeDtypeStruct((P, O), jnp.float32),
    )(e, Wc1a, Wc1b, bc1.reshape(1, D), Wc2, bc2.reshape(1, H2), Wc3,
      bc3.reshape(1, O))


def kernel(x, edge_index, drug_pairs, W0, b0, g0, beta0, W1, b1, g1, beta1,
           Wc1, bc1, Wc2, bc2, Wc3, bc3):
    f32 = jnp.float32
    i32 = jnp.int32
    zeros2d = jnp.zeros((RPS, D), f32)
    pad_p = jnp.arange(PP - P, dtype=i32) % N
    pairs_pad = jnp.concatenate(
        [drug_pairs.T.astype(i32), jnp.stack([pad_p, pad_p])], axis=1
    ).reshape(2, NW, PCH, 128)
    pairs = jnp.concatenate([pairs_pad[0], pairs_pad[1]], axis=1)

    dego = _deg_call(edge_index)
    h0 = _mm0_call(x, W0)
    hs1, dinv = _scale_call(h0, dego)
    acc1 = _spmm_call(hs1, edge_index, zeros2d)
    hs2 = _bn_mm_call(acc1, hs1, dinv, b0, g0, beta0, W1)
    acc2 = _spmm_call(hs2, edge_index, zeros2d)
    hfin = _bn_call(acc2, hs2, dinv, b1, g1, beta1)
    e = _pair_call(hfin, pairs)
    return _mlp_call(e, Wc1[:D], Wc1[D:], bc1, Wc2, bc2, Wc3, bc3)
